# Initial kernel scaffold; baseline (speedup 1.0000x reference)
import jax
import jax.numpy as jnp
from jax.experimental import pallas as pl


def kernel(x, edge_index, W1, att_src1, att_dst1, b1, W2, att_src2, att_dst2, b2):
    raise NotImplementedError("write your pallas kernel here")


# speedup vs baseline: 65.3588x; 65.3588x over previous
import functools

import jax
import jax.numpy as jnp
from jax import lax
from jax.experimental import pallas as pl
from jax.experimental.pallas import tpu as pltpu
from jax.experimental.pallas import tpu_sc as plsc

N = 10000
F_IN = 128
H1 = 8
D1 = 8
DH1 = 64
C = 10
DH2 = 16

NPAD = 10240
E = 320000
ET = E + N
NC = 2
NS = 16
NW = NC * NS
B = 128
ITERS = -(-ET // (NW * B))
ETP = NW * B * ITERS
ROWS_PER_TILE = NPAD // NS


def _sc_edge_sweep(DH, D):
    CH = DH // 16
    mesh = plsc.VectorSubcoreMesh(
        core_axis_name="c", subcore_axis_name="s",
        num_cores=NC, num_subcores=NS)

    @functools.partial(
        pl.kernel,
        out_type=(
            jax.ShapeDtypeStruct((NC, NPAD, DH), jnp.float32),
            jax.ShapeDtypeStruct((NC, NPAD, 16), jnp.float32),
        ),
        mesh=mesh,
        compiler_params=pltpu.CompilerParams(use_tc_tiling_on_sc=False),
        scratch_types=[
            pltpu.VMEM((B,), jnp.int32),
            pltpu.VMEM((B,), jnp.int32),
            pltpu.VMEM((B, 16), jnp.float32),
            pltpu.VMEM((B, 16), jnp.float32),
            pltpu.VMEM((B, 16), jnp.float32),
            pltpu.VMEM((B, DH), jnp.float32),
            pltpu.VMEM((B, DH), jnp.float32),
            pltpu.VMEM_SHARED((NPAD, DH), jnp.float32),
            pltpu.VMEM_SHARED((NPAD, 16), jnp.float32),
            pltpu.SemaphoreType.DMA,
            pltpu.SemaphoreType.DMA,
            pltpu.SemaphoreType.DMA,
        ],
    )
    def sweep(src_hbm, dst_hbm, htab, astab, adtab,
              onum, oden,
              src_v, dst_v, as_v, ad_v, w_v, h_v, msg_v,
              num_sh, den_sh, sem0, sem1, sem2):
        cid = lax.axis_index("c")
        sid = lax.axis_index("s")
        wid = cid * NS + sid

        zero16 = jnp.zeros((16,), jnp.float32)

        def zrow(e, carry):
            for c in range(CH):
                msg_v[e, pl.ds(c * 16, 16)] = zero16
            w_v[e, :] = zero16
            return carry

        lax.fori_loop(0, B, zrow, 0)

        r0 = sid * ROWS_PER_TILE

        def zacc(k, carry):
            pltpu.sync_copy(msg_v, num_sh.at[pl.ds(r0 + k * B, B)])
            pltpu.sync_copy(w_v, den_sh.at[pl.ds(r0 + k * B, B)])
            return carry

        lax.fori_loop(0, ROWS_PER_TILE // B, zacc, 0)
        plsc.subcore_barrier()

        iota16 = lax.iota(jnp.int32, 16)
        low8 = iota16 < 8
        base0 = wid * (ITERS * B)

        def block(i, carry):
            base = base0 + i * B
            pltpu.sync_copy(src_hbm.at[pl.ds(base, B)], src_v)
            pltpu.sync_copy(dst_hbm.at[pl.ds(base, B)], dst_v)
            ca = pltpu.async_copy(astab.at[src_v], as_v, sem0)
            cb = pltpu.async_copy(adtab.at[dst_v], ad_v, sem1)
            cc = pltpu.async_copy(htab.at[src_v], h_v, sem2)
            ca.wait()
            cb.wait()

            def wbody(e, carry2):
                a = as_v[e, :] + ad_v[e, :]
                a = jnp.where(a >= 0.0, a, a * 0.2)
                w_v[e, :] = jnp.exp(a)
                return carry2

            lax.fori_loop(0, B, wbody, 0)
            pltpu.sync_copy(w_v, den_sh.at[dst_v], add=True)
            cc.wait()

            def mbody(e, carry2):
                wrow = w_v[e, :]
                for c in range(CH):
                    ha = (c * 16) // D
                    hb = (c * 16 + 8) // D
                    wv = jnp.where(low8,
                                   jnp.full((16,), wrow[ha]),
                                   jnp.full((16,), wrow[hb]))
                    msg_v[e, pl.ds(c * 16, 16)] = (
                        h_v[e, pl.ds(c * 16, 16)] * wv)
                return carry2

            lax.fori_loop(0, B, mbody, 0)
            pltpu.sync_copy(msg_v, num_sh.at[dst_v], add=True)
            return carry

        lax.fori_loop(0, ITERS, block, 0)
        plsc.subcore_barrier()
        pltpu.sync_copy(num_sh.at[pl.ds(r0, ROWS_PER_TILE)],
                        onum.at[cid, pl.ds(r0, ROWS_PER_TILE)])
        pltpu.sync_copy(den_sh.at[pl.ds(r0, ROWS_PER_TILE)],
                        oden.at[cid, pl.ds(r0, ROWS_PER_TILE)])

    return sweep


_sweep1 = _sc_edge_sweep(DH1, D1)
_sweep2 = _sc_edge_sweep(DH2, DH2)


def _tc_layer1(xp, W1, asw, adw):
    def body(x_ref, w_ref, as_ref, ad_ref, h_ref, s_ref, d_ref):
        h = jnp.dot(x_ref[...], w_ref[...],
                    preferred_element_type=jnp.float32)
        h_ref[...] = h
        s_ref[...] = jnp.dot(h, as_ref[...],
                             preferred_element_type=jnp.float32)
        d_ref[...] = jnp.dot(h, ad_ref[...],
                             preferred_element_type=jnp.float32)

    return pl.pallas_call(
        body,
        out_shape=(
            jax.ShapeDtypeStruct((NPAD, DH1), jnp.float32),
            jax.ShapeDtypeStruct((NPAD, 16), jnp.float32),
            jax.ShapeDtypeStruct((NPAD, 16), jnp.float32),
        ),
    )(xp, W1, asw, adw)


def _tc_combine1(onum, oden, b1r, W2p, as2w, ad2w, rmat):
    def body(on_ref, od_ref, b1_ref, w2_ref, as_ref, ad_ref, r_ref,
             h2_ref, s2_ref, d2_ref):
        o = on_ref[0] + on_ref[1]
        den = od_ref[0] + od_ref[1]
        inv = 1.0 / (den + 1e-16)
        invb = jnp.dot(inv, r_ref[...], preferred_element_type=jnp.float32)
        out1 = o * invb + b1_ref[...]
        x2 = jnp.where(out1 > 0.0, out1, jnp.exp(out1) - 1.0)
        h2 = jnp.dot(x2, w2_ref[...], preferred_element_type=jnp.float32)
        h2_ref[...] = h2
        s2_ref[...] = jnp.dot(h2, as_ref[...],
                              preferred_element_type=jnp.float32)
        d2_ref[...] = jnp.dot(h2, ad_ref[...],
                              preferred_element_type=jnp.float32)

    return pl.pallas_call(
        body,
        out_shape=(
            jax.ShapeDtypeStruct((NPAD, DH2), jnp.float32),
            jax.ShapeDtypeStruct((NPAD, 16), jnp.float32),
            jax.ShapeDtypeStruct((NPAD, 16), jnp.float32),
        ),
    )(onum, oden, b1r, W2p, as2w, ad2w, rmat)


def _tc_final(onum, oden, b2r):
    def body(on_ref, od_ref, b2_ref, out_ref):
        o = on_ref[0] + on_ref[1]
        den = od_ref[0] + od_ref[1]
        col = lax.broadcasted_iota(jnp.int32, (NPAD, 16), 1)
        d0 = jnp.sum(jnp.where(col < 1, den, 0.0), axis=1, keepdims=True)
        logits = o * (1.0 / (d0 + 1e-16)) + b2_ref[...]
        valid = col < C
        masked = jnp.where(valid, logits, -jnp.inf)
        m = jnp.max(masked, axis=1, keepdims=True)
        z = logits - m
        ez = jnp.where(valid, jnp.exp(z), 0.0)
        s = jnp.sum(ez, axis=1, keepdims=True)
        out_ref[...] = z - jnp.log(s)

    return pl.pallas_call(
        body,
        out_shape=jax.ShapeDtypeStruct((NPAD, 16), jnp.float32),
    )(onum, oden, b2r)


def kernel(x, edge_index, W1, att_src1, att_dst1, b1,
           W2, att_src2, att_dst2, b2):
    loop = jnp.arange(N, dtype=jnp.int32)
    npadedge = ETP - ET
    src = jnp.concatenate([edge_index[0].astype(jnp.int32), loop,
                           jnp.zeros((npadedge,), jnp.int32)])
    dst = jnp.concatenate([edge_index[1].astype(jnp.int32), loop,
                           jnp.full((npadedge,), N, jnp.int32)])

    xp = jnp.zeros((NPAD, F_IN), jnp.float32).at[:N].set(x)

    j = jnp.arange(DH1)
    asw = jnp.zeros((DH1, 16), jnp.float32).at[j, j // D1].set(
        att_src1.reshape(-1))
    adw = jnp.zeros((DH1, 16), jnp.float32).at[j, j // D1].set(
        att_dst1.reshape(-1))
    rmat = jnp.zeros((16, DH1), jnp.float32).at[j // D1, j].set(1.0)
    W2p = jnp.zeros((DH1, 16), jnp.float32).at[:, :C].set(W2)
    as2w = jnp.zeros((16, 16), jnp.float32).at[:C, 0].set(att_src2[0])
    ad2w = jnp.zeros((16, 16), jnp.float32).at[:C, 0].set(att_dst2[0])
    b1r = b1.reshape(1, DH1)
    b2r = jnp.zeros((1, 16), jnp.float32).at[0, :C].set(b2)

    h1, as1, ad1 = _tc_layer1(xp, W1, asw, adw)
    onum1, oden1 = _sweep1(src, dst, h1, as1, ad1)

    h2, as2, ad2 = _tc_combine1(onum1, oden1, b1r, W2p, as2w, ad2w, rmat)
    onum2, oden2 = _sweep2(src, dst, h2, as2, ad2)

    out = _tc_final(onum2, oden2, b2r)
    return out[:N, :C]


# speedup vs baseline: 78.3343x; 1.1985x over previous
import functools

import jax
import jax.numpy as jnp
from jax import lax
from jax.experimental import pallas as pl
from jax.experimental.pallas import tpu as pltpu
from jax.experimental.pallas import tpu_sc as plsc

N = 10000
F_IN = 128
H1 = 8
D1 = 8
DH1 = 64
C = 10
DH2 = 16

NPAD = 10240
E = 320000
ET = E + N
NC = 2
NS = 16
NW = NC * NS
B = 128
ITERS = 4 * (-(-ET // (NW * B * 4)))
ETP = NW * B * ITERS
EARR = ETP + 2 * B
ROWS_PER_TILE = NPAD // NS


def _sc_edge_sweep(DH):
    CH = DH // 16
    mesh = plsc.VectorSubcoreMesh(
        core_axis_name="c", subcore_axis_name="s",
        num_cores=NC, num_subcores=NS)

    @functools.partial(
        pl.kernel,
        out_type=(
            jax.ShapeDtypeStruct((NC, NPAD, DH), jnp.float32),
            jax.ShapeDtypeStruct((NC, NPAD, 16), jnp.float32),
        ),
        mesh=mesh,
        compiler_params=pltpu.CompilerParams(use_tc_tiling_on_sc=False),
        scratch_types=[
            pltpu.VMEM((4, B), jnp.int32),
            pltpu.VMEM((4, B), jnp.int32),
            pltpu.VMEM((2, B, 16), jnp.float32),
            pltpu.VMEM((2, B, 16), jnp.float32),
            pltpu.VMEM((2, B, 16), jnp.float32),
            pltpu.VMEM((2, B, DH), jnp.float32),
            pltpu.VMEM((2, B, DH), jnp.float32),
            pltpu.VMEM_SHARED((NPAD, DH), jnp.float32),
            pltpu.VMEM_SHARED((NPAD, 16), jnp.float32),
            [pltpu.SemaphoreType.DMA] * 4,
            [pltpu.SemaphoreType.DMA] * 2,
            [pltpu.SemaphoreType.DMA] * 2,
        ],
    )
    def sweep(src_hbm, dst_hbm, htab, astab, adtab,
              onum, oden,
              sidx, didx, as_v, ad_v, w_v, h_v, msg_v,
              num_sh, den_sh, isem, gsem, ssem):
        cid = lax.axis_index("c")
        sid = lax.axis_index("s")
        wid = cid * NS + sid
        base0 = wid * (ITERS * B)

        zero16 = jnp.zeros((16,), jnp.float32)

        def zrow(e, carry):
            for c in range(CH):
                msg_v[0, e, pl.ds(c * 16, 16)] = zero16
            w_v[0, e, :] = zero16
            return carry

        lax.fori_loop(0, B, zrow, 0)
        r0 = sid * ROWS_PER_TILE

        def zacc(q, carry):
            pltpu.sync_copy(msg_v.at[0], num_sh.at[pl.ds(r0 + q * B, B)])
            pltpu.sync_copy(w_v.at[0], den_sh.at[pl.ds(r0 + q * B, B)])
            return carry

        lax.fori_loop(0, ROWS_PER_TILE // B, zacc, 0)
        plsc.subcore_barrier()

        def issue_idx(i, s):
            pltpu.async_copy(src_hbm.at[pl.ds(base0 + i * B, B)],
                             sidx.at[s], isem[s])
            pltpu.async_copy(dst_hbm.at[pl.ds(base0 + i * B, B)],
                             didx.at[s], isem[s])

        def wait_idx(s):
            pltpu.make_async_copy(src_hbm.at[pl.ds(0, B)],
                                  sidx.at[s], isem[s]).wait()
            pltpu.make_async_copy(dst_hbm.at[pl.ds(0, B)],
                                  didx.at[s], isem[s]).wait()

        def issue_gath(x, s):
            pltpu.async_copy(astab.at[sidx.at[s]], as_v.at[x], gsem[x])
            pltpu.async_copy(adtab.at[didx.at[s]], ad_v.at[x], gsem[x])
            pltpu.async_copy(htab.at[sidx.at[s]], h_v.at[x], gsem[x])

        def wait_gath(x, s):
            pltpu.make_async_copy(astab.at[sidx.at[s]],
                                  as_v.at[x], gsem[x]).wait()
            pltpu.make_async_copy(adtab.at[didx.at[s]],
                                  ad_v.at[x], gsem[x]).wait()
            pltpu.make_async_copy(htab.at[sidx.at[s]],
                                  h_v.at[x], gsem[x]).wait()

        def issue_scat(x, s):
            pltpu.async_copy(w_v.at[x], den_sh.at[didx.at[s]], ssem[x],
                             add=True)
            pltpu.async_copy(msg_v.at[x], num_sh.at[didx.at[s]], ssem[x],
                             add=True)

        def wait_scat(x, s):
            pltpu.make_async_copy(w_v.at[x], den_sh.at[didx.at[s]],
                                  ssem[x]).wait()
            pltpu.make_async_copy(msg_v.at[x], num_sh.at[didx.at[s]],
                                  ssem[x]).wait()

        def compute(x):
            def body(e, carry):
                a = as_v[x, e, :] + ad_v[x, e, :]
                a = jnp.where(a >= 0.0, a, a * 0.2)
                w16 = jnp.exp(a)
                w_v[x, e, :] = w16
                for c in range(CH):
                    msg_v[x, e, pl.ds(c * 16, 16)] = (
                        h_v[x, e, pl.ds(c * 16, 16)] * w16)
                return carry
            lax.fori_loop(0, B, body, 0)

        issue_idx(0, 0)
        issue_idx(1, 1)
        wait_idx(0)
        issue_gath(0, 0)

        def quad(k, carry):
            i0 = 4 * k
            for p in range(4):
                x = p % 2
                y = 1 - x
                sp = p
                sp1 = (p + 1) % 4
                sp2 = (p + 2) % 4
                if p < 2:
                    @pl.when(k >= 1)
                    def _():
                        wait_scat(x, sp2)
                else:
                    wait_scat(x, sp2)
                issue_idx(i0 + p + 2, sp2)
                wait_idx(sp1)
                issue_gath(y, sp1)
                wait_gath(x, sp)
                compute(x)
                issue_scat(x, sp)
            return carry

        lax.fori_loop(0, ITERS // 4, quad, 0)

        wait_scat(0, 2)
        wait_scat(1, 3)
        wait_gath(0, 0)
        wait_idx(1)

        plsc.subcore_barrier()
        pltpu.sync_copy(num_sh.at[pl.ds(r0, ROWS_PER_TILE)],
                        onum.at[cid, pl.ds(r0, ROWS_PER_TILE)])
        pltpu.sync_copy(den_sh.at[pl.ds(r0, ROWS_PER_TILE)],
                        oden.at[cid, pl.ds(r0, ROWS_PER_TILE)])

    return sweep


_sweep1 = _sc_edge_sweep(DH1)
_sweep2 = _sc_edge_sweep(DH2)


def _tc_layer1(xp, W1p, asw, adw):
    def body(x_ref, w_ref, as_ref, ad_ref, h_ref, s_ref, d_ref):
        h = jnp.dot(x_ref[...], w_ref[...],
                    preferred_element_type=jnp.float32)
        h_ref[...] = h
        s_ref[...] = jnp.dot(h, as_ref[...],
                             preferred_element_type=jnp.float32)
        d_ref[...] = jnp.dot(h, ad_ref[...],
                             preferred_element_type=jnp.float32)

    return pl.pallas_call(
        body,
        out_shape=(
            jax.ShapeDtypeStruct((NPAD, DH1), jnp.float32),
            jax.ShapeDtypeStruct((NPAD, 16), jnp.float32),
            jax.ShapeDtypeStruct((NPAD, 16), jnp.float32),
        ),
    )(xp, W1p, asw, adw)


def _tc_combine1(onum, oden, b1r, W2p, as2w, ad2w, rmat):
    def body(on_ref, od_ref, b1_ref, w2_ref, as_ref, ad_ref, r_ref,
             h2_ref, s2_ref, d2_ref):
        o = on_ref[0] + on_ref[1]
        den = od_ref[0] + od_ref[1]
        inv = 1.0 / (den + 1e-16)
        invb = jnp.dot(inv, r_ref[...], preferred_element_type=jnp.float32)
        out1 = o * invb + b1_ref[...]
        x2 = jnp.where(out1 > 0.0, out1, jnp.exp(out1) - 1.0)
        h2 = jnp.dot(x2, w2_ref[...], preferred_element_type=jnp.float32)
        h2_ref[...] = h2
        s2_ref[...] = jnp.dot(h2, as_ref[...],
                              preferred_element_type=jnp.float32)
        d2_ref[...] = jnp.dot(h2, ad_ref[...],
                              preferred_element_type=jnp.float32)

    return pl.pallas_call(
        body,
        out_shape=(
            jax.ShapeDtypeStruct((NPAD, DH2), jnp.float32),
            jax.ShapeDtypeStruct((NPAD, 16), jnp.float32),
            jax.ShapeDtypeStruct((NPAD, 16), jnp.float32),
        ),
    )(onum, oden, b1r, W2p, as2w, ad2w, rmat)


def _tc_final(onum, oden, b2r):
    def body(on_ref, od_ref, b2_ref, out_ref):
        o = on_ref[0] + on_ref[1]
        den = od_ref[0] + od_ref[1]
        col = lax.broadcasted_iota(jnp.int32, (NPAD, 16), 1)
        d0 = jnp.sum(jnp.where(col < 1, den, 0.0), axis=1, keepdims=True)
        logits = o * (1.0 / (d0 + 1e-16)) + b2_ref[...]
        valid = col < C
        masked = jnp.where(valid, logits, -jnp.inf)
        m = jnp.max(masked, axis=1, keepdims=True)
        z = logits - m
        ez = jnp.where(valid, jnp.exp(z), 0.0)
        s = jnp.sum(ez, axis=1, keepdims=True)
        out_ref[...] = z - jnp.log(s)

    return pl.pallas_call(
        body,
        out_shape=jax.ShapeDtypeStruct((NPAD, 16), jnp.float32),
    )(onum, oden, b2r)


def kernel(x, edge_index, W1, att_src1, att_dst1, b1,
           W2, att_src2, att_dst2, b2):
    loop = jnp.arange(N, dtype=jnp.int32)
    npadedge = EARR - ET
    src = jnp.concatenate([edge_index[0].astype(jnp.int32), loop,
                           jnp.zeros((npadedge,), jnp.int32)])
    dst = jnp.concatenate([edge_index[1].astype(jnp.int32), loop,
                           jnp.full((npadedge,), N, jnp.int32)])

    xp = jnp.zeros((NPAD, F_IN), jnp.float32).at[:N].set(x)

    j = jnp.arange(DH1)
    perm = (j % H1) * D1 + j // H1
    W1p = W1[:, perm]
    hh = j % H1
    dd = j // H1
    kk = jnp.arange(16)
    sel = (hh[:, None] == (kk[None, :] % H1)).astype(jnp.float32)
    asw = att_src1[hh, dd][:, None] * sel
    adw = att_dst1[hh, dd][:, None] * sel
    rmat = jnp.zeros((16, DH1), jnp.float32).at[j % H1, j].set(1.0)
    b1r = b1[perm].reshape(1, DH1)
    W2p = jnp.zeros((DH1, 16), jnp.float32).at[:, :C].set(W2[perm, :])
    as2w = jnp.zeros((16, 16), jnp.float32).at[:C, :].set(
        jnp.broadcast_to(att_src2[0][:, None], (C, 16)))
    ad2w = jnp.zeros((16, 16), jnp.float32).at[:C, :].set(
        jnp.broadcast_to(att_dst2[0][:, None], (C, 16)))
    b2r = jnp.zeros((1, 16), jnp.float32).at[0, :C].set(b2)

    h1, as1, ad1 = _tc_layer1(xp, W1p, asw, adw)
    onum1, oden1 = _sweep1(src, dst, h1, as1, ad1)

    h2, as2, ad2 = _tc_combine1(onum1, oden1, b1r, W2p, as2w, ad2w, rmat)
    onum2, oden2 = _sweep2(src, dst, h2, as2, ad2)

    out = _tc_final(onum2, oden2, b2r)
    return out[:N, :C]


# speedup vs baseline: 83.9652x; 1.0719x over previous
import functools

import jax
import jax.numpy as jnp
from jax import lax
from jax.experimental import pallas as pl
from jax.experimental.pallas import tpu as pltpu
from jax.experimental.pallas import tpu_sc as plsc

N = 10000
F_IN = 128
H1 = 8
D1 = 8
DH1 = 64
C = 10
DH2 = 16

NPAD = 10240
E = 320000
ET = E + N
NC = 2
NS = 16
NW = NC * NS
B = 128
ITERS = 4 * (-(-ET // (NW * B * 4)))
ETP = NW * B * ITERS
EARR = ETP + 2 * B
ROWS_PER_TILE = NPAD // NS


def _sc_edge_sweep(DH):
    CH = DH // 16
    mesh = plsc.VectorSubcoreMesh(
        core_axis_name="c", subcore_axis_name="s",
        num_cores=NC, num_subcores=NS)

    @functools.partial(
        pl.kernel,
        out_type=(
            jax.ShapeDtypeStruct((NC, NPAD, DH), jnp.float32),
            jax.ShapeDtypeStruct((NC, NPAD, 16), jnp.float32),
        ),
        mesh=mesh,
        compiler_params=pltpu.CompilerParams(use_tc_tiling_on_sc=False),
        scratch_types=[
            pltpu.VMEM((4, B), jnp.int32),
            pltpu.VMEM((4, B), jnp.int32),
            pltpu.VMEM((2, B, 16), jnp.float32),
            pltpu.VMEM((2, B, 16), jnp.float32),
            pltpu.VMEM((2, B, 16), jnp.float32),
            pltpu.VMEM((2, B, DH), jnp.float32),
            pltpu.VMEM((2, B, DH), jnp.float32),
            pltpu.VMEM_SHARED((NPAD, DH), jnp.float32),
            pltpu.VMEM_SHARED((NPAD, 16), jnp.float32),
            [pltpu.SemaphoreType.DMA] * 4,
            [pltpu.SemaphoreType.DMA] * 2,
            [pltpu.SemaphoreType.DMA] * 2,
        ],
    )
    def sweep(src_hbm, dst_hbm, htab, astab, adtab,
              onum, oden,
              sidx, didx, as_v, ad_v, w_v, h_v, msg_v,
              num_sh, den_sh, isem, gsem, ssem):
        cid = lax.axis_index("c")
        sid = lax.axis_index("s")
        wid = cid * NS + sid
        base0 = wid * (ITERS * B)

        zero16 = jnp.zeros((16,), jnp.float32)

        def zrow(e, carry):
            for c in range(CH):
                msg_v[0, e, pl.ds(c * 16, 16)] = zero16
            w_v[0, e, :] = zero16
            return carry

        lax.fori_loop(0, B, zrow, 0)
        r0 = sid * ROWS_PER_TILE

        def zacc(q, carry):
            pltpu.sync_copy(msg_v.at[0], num_sh.at[pl.ds(r0 + q * B, B)])
            pltpu.sync_copy(w_v.at[0], den_sh.at[pl.ds(r0 + q * B, B)])
            return carry

        lax.fori_loop(0, ROWS_PER_TILE // B, zacc, 0)
        plsc.subcore_barrier()

        def issue_idx(i, s):
            pltpu.async_copy(src_hbm.at[pl.ds(base0 + i * B, B)],
                             sidx.at[s], isem[s])
            pltpu.async_copy(dst_hbm.at[pl.ds(base0 + i * B, B)],
                             didx.at[s], isem[s])

        def wait_idx(s):
            pltpu.make_async_copy(src_hbm.at[pl.ds(0, B)],
                                  sidx.at[s], isem[s]).wait()
            pltpu.make_async_copy(dst_hbm.at[pl.ds(0, B)],
                                  didx.at[s], isem[s]).wait()

        def issue_gath(x, s):
            pltpu.async_copy(astab.at[sidx.at[s]], as_v.at[x], gsem[x])
            pltpu.async_copy(adtab.at[didx.at[s]], ad_v.at[x], gsem[x])
            pltpu.async_copy(htab.at[sidx.at[s]], h_v.at[x], gsem[x])

        def wait_gath(x, s):
            pltpu.make_async_copy(astab.at[sidx.at[s]],
                                  as_v.at[x], gsem[x]).wait()
            pltpu.make_async_copy(adtab.at[didx.at[s]],
                                  ad_v.at[x], gsem[x]).wait()
            pltpu.make_async_copy(htab.at[sidx.at[s]],
                                  h_v.at[x], gsem[x]).wait()

        def issue_scat(x, s):
            pltpu.async_copy(w_v.at[x], den_sh.at[didx.at[s]], ssem[x],
                             add=True)
            pltpu.async_copy(msg_v.at[x], num_sh.at[didx.at[s]], ssem[x],
                             add=True)

        def wait_scat(x, s):
            pltpu.make_async_copy(w_v.at[x], den_sh.at[didx.at[s]],
                                  ssem[x]).wait()
            pltpu.make_async_copy(msg_v.at[x], num_sh.at[didx.at[s]],
                                  ssem[x]).wait()

        def compute(x):
            @plsc.parallel_loop(0, B, unroll=4)
            def _(e):
                a = as_v[x, e, :] + ad_v[x, e, :]
                a = jnp.where(a >= 0.0, a, a * 0.2)
                w16 = jnp.exp(a)
                w_v[x, e, :] = w16
                for c in range(CH):
                    msg_v[x, e, pl.ds(c * 16, 16)] = (
                        h_v[x, e, pl.ds(c * 16, 16)] * w16)

        issue_idx(0, 0)
        issue_idx(1, 1)
        wait_idx(0)
        issue_gath(0, 0)

        def quad(k, carry):
            i0 = 4 * k
            for p in range(4):
                x = p % 2
                y = 1 - x
                sp = p
                sp1 = (p + 1) % 4
                sp2 = (p + 2) % 4
                if p < 2:
                    @pl.when(k >= 1)
                    def _():
                        wait_scat(x, sp2)
                else:
                    wait_scat(x, sp2)
                issue_idx(i0 + p + 2, sp2)
                wait_idx(sp1)
                issue_gath(y, sp1)
                wait_gath(x, sp)
                compute(x)
                issue_scat(x, sp)
            return carry

        lax.fori_loop(0, ITERS // 4, quad, 0)

        wait_scat(0, 2)
        wait_scat(1, 3)
        wait_gath(0, 0)
        wait_idx(1)

        plsc.subcore_barrier()
        pltpu.sync_copy(num_sh.at[pl.ds(r0, ROWS_PER_TILE)],
                        onum.at[cid, pl.ds(r0, ROWS_PER_TILE)])
        pltpu.sync_copy(den_sh.at[pl.ds(r0, ROWS_PER_TILE)],
                        oden.at[cid, pl.ds(r0, ROWS_PER_TILE)])

    return sweep


_sweep1 = _sc_edge_sweep(DH1)
_sweep2 = _sc_edge_sweep(DH2)


def _tc_layer1(xp, W1p, asw, adw):
    def body(x_ref, w_ref, as_ref, ad_ref, h_ref, s_ref, d_ref):
        h = jnp.dot(x_ref[...], w_ref[...],
                    preferred_element_type=jnp.float32)
        h_ref[...] = h
        s_ref[...] = jnp.dot(h, as_ref[...],
                             preferred_element_type=jnp.float32)
        d_ref[...] = jnp.dot(h, ad_ref[...],
                             preferred_element_type=jnp.float32)

    return pl.pallas_call(
        body,
        out_shape=(
            jax.ShapeDtypeStruct((NPAD, DH1), jnp.float32),
            jax.ShapeDtypeStruct((NPAD, 16), jnp.float32),
            jax.ShapeDtypeStruct((NPAD, 16), jnp.float32),
        ),
    )(xp, W1p, asw, adw)


def _tc_combine1(onum, oden, b1r, W2p, as2w, ad2w, rmat):
    def body(on_ref, od_ref, b1_ref, w2_ref, as_ref, ad_ref, r_ref,
             h2_ref, s2_ref, d2_ref):
        o = on_ref[0] + on_ref[1]
        den = od_ref[0] + od_ref[1]
        inv = 1.0 / (den + 1e-16)
        invb = jnp.dot(inv, r_ref[...], preferred_element_type=jnp.float32)
        out1 = o * invb + b1_ref[...]
        x2 = jnp.where(out1 > 0.0, out1, jnp.exp(out1) - 1.0)
        h2 = jnp.dot(x2, w2_ref[...], preferred_element_type=jnp.float32)
        h2_ref[...] = h2
        s2_ref[...] = jnp.dot(h2, as_ref[...],
                              preferred_element_type=jnp.float32)
        d2_ref[...] = jnp.dot(h2, ad_ref[...],
                              preferred_element_type=jnp.float32)

    return pl.pallas_call(
        body,
        out_shape=(
            jax.ShapeDtypeStruct((NPAD, DH2), jnp.float32),
            jax.ShapeDtypeStruct((NPAD, 16), jnp.float32),
            jax.ShapeDtypeStruct((NPAD, 16), jnp.float32),
        ),
    )(onum, oden, b1r, W2p, as2w, ad2w, rmat)


def _tc_final(onum, oden, b2r):
    def body(on_ref, od_ref, b2_ref, out_ref):
        o = on_ref[0] + on_ref[1]
        den = od_ref[0] + od_ref[1]
        col = lax.broadcasted_iota(jnp.int32, (NPAD, 16), 1)
        d0 = jnp.sum(jnp.where(col < 1, den, 0.0), axis=1, keepdims=True)
        logits = o * (1.0 / (d0 + 1e-16)) + b2_ref[...]
        valid = col < C
        masked = jnp.where(valid, logits, -jnp.inf)
        m = jnp.max(masked, axis=1, keepdims=True)
        z = logits - m
        ez = jnp.where(valid, jnp.exp(z), 0.0)
        s = jnp.sum(ez, axis=1, keepdims=True)
        out_ref[...] = z - jnp.log(s)

    return pl.pallas_call(
        body,
        out_shape=jax.ShapeDtypeStruct((NPAD, 16), jnp.float32),
    )(onum, oden, b2r)


def kernel(x, edge_index, W1, att_src1, att_dst1, b1,
           W2, att_src2, att_dst2, b2):
    loop = jnp.arange(N, dtype=jnp.int32)
    npadedge = EARR - ET
    src = jnp.concatenate([edge_index[0].astype(jnp.int32), loop,
                           jnp.zeros((npadedge,), jnp.int32)])
    dst = jnp.concatenate([edge_index[1].astype(jnp.int32), loop,
                           jnp.full((npadedge,), N, jnp.int32)])

    xp = jnp.zeros((NPAD, F_IN), jnp.float32).at[:N].set(x)

    j = jnp.arange(DH1)
    perm = (j % H1) * D1 + j // H1
    W1p = W1[:, perm]
    hh = j % H1
    dd = j // H1
    kk = jnp.arange(16)
    sel = (hh[:, None] == (kk[None, :] % H1)).astype(jnp.float32)
    asw = att_src1[hh, dd][:, None] * sel
    adw = att_dst1[hh, dd][:, None] * sel
    rmat = jnp.zeros((16, DH1), jnp.float32).at[j % H1, j].set(1.0)
    b1r = b1[perm].reshape(1, DH1)
    W2p = jnp.zeros((DH1, 16), jnp.float32).at[:, :C].set(W2[perm, :])
    as2w = jnp.zeros((16, 16), jnp.float32).at[:C, :].set(
        jnp.broadcast_to(att_src2[0][:, None], (C, 16)))
    ad2w = jnp.zeros((16, 16), jnp.float32).at[:C, :].set(
        jnp.broadcast_to(att_dst2[0][:, None], (C, 16)))
    b2r = jnp.zeros((1, 16), jnp.float32).at[0, :C].set(b2)

    h1, as1, ad1 = _tc_layer1(xp, W1p, asw, adw)
    onum1, oden1 = _sweep1(src, dst, h1, as1, ad1)

    h2, as2, ad2 = _tc_combine1(onum1, oden1, b1r, W2p, as2w, ad2w, rmat)
    onum2, oden2 = _sweep2(src, dst, h2, as2, ad2)

    out = _tc_final(onum2, oden2, b2r)
    return out[:N, :C]


# speedup vs baseline: 88.2958x; 1.0516x over previous
import functools

import jax
import jax.numpy as jnp
from jax import lax
from jax.experimental import pallas as pl
from jax.experimental.pallas import tpu as pltpu
from jax.experimental.pallas import tpu_sc as plsc

N = 10000
F_IN = 128
H1 = 8
D1 = 8
DH1 = 64
C = 10
DH2 = 16

NPAD = 10240
E = 320000
ET = E + N
NC = 2
NS = 16
NW = NC * NS
B = 128
TBLK = 168
ETP = NS * B * TBLK
EARR = ETP + 2 * B
ROWS_PER_TILE = NPAD // NS


def _sc_edge_sweep(DH, den_in_msg=False, stage_feat=True, it0=TBLK // 2, it1=TBLK // 2):
    CH = DH // 16
    FW = DH
    mesh = plsc.VectorSubcoreMesh(
        core_axis_name="c", subcore_axis_name="s",
        num_cores=NC, num_subcores=NS)

    out_type = [jax.ShapeDtypeStruct((NC, NPAD, DH), jnp.float32)]
    if not den_in_msg:
        out_type.append(jax.ShapeDtypeStruct((NC, NPAD, 16), jnp.float32))
    scr = [
        pltpu.VMEM((4, B), jnp.int32),
        pltpu.VMEM((4, B), jnp.int32),
        pltpu.VMEM((2, B, 16), jnp.float32),
        pltpu.VMEM((2, B, 16), jnp.float32),
        pltpu.VMEM((2, B, 16), jnp.float32),
        pltpu.VMEM((2, B, FW), jnp.float32),
        pltpu.VMEM((2, B, DH), jnp.float32),
        pltpu.VMEM_SHARED((NPAD, DH), jnp.float32),
    ]
    if not den_in_msg:
        scr.append(pltpu.VMEM_SHARED((NPAD, 16), jnp.float32))
    if stage_feat:
        scr.append(pltpu.VMEM_SHARED((NPAD, FW), jnp.float32))
    scr += [
        [pltpu.SemaphoreType.DMA] * 4,
        [pltpu.SemaphoreType.DMA] * 2,
        [pltpu.SemaphoreType.DMA] * 2,
    ]

    @functools.partial(
        pl.kernel,
        out_type=tuple(out_type) if len(out_type) > 1 else out_type[0],
        mesh=mesh,
        compiler_params=pltpu.CompilerParams(use_tc_tiling_on_sc=False),
        scratch_types=scr,
    )
    def sweep(src_hbm, dst_hbm, htab, astab, adtab, *rest):
        rest = list(rest)
        onum = rest.pop(0)
        oden = None if den_in_msg else rest.pop(0)
        sidx, didx, as_v, ad_v, w_v, h_v, msg_v, num_sh = rest[:8]
        rest = rest[8:]
        den_sh = None if den_in_msg else rest.pop(0)
        htab_sh = rest.pop(0) if stage_feat else htab
        isem, gsem, ssem = rest[-3:]
        cid = lax.axis_index("c")
        sid = lax.axis_index("s")
        itc = jnp.where(cid == 0, it0, it1)
        base0 = jnp.where(cid == 0, sid * (it0 * B),
                          NS * it0 * B + sid * (it1 * B))

        zero16 = jnp.zeros((16,), jnp.float32)

        def zrow(e, carry):
            for c in range(CH):
                msg_v[0, e, pl.ds(c * 16, 16)] = zero16
            if not den_in_msg:
                w_v[0, e, :] = zero16
            return carry

        lax.fori_loop(0, B, zrow, 0)
        r0 = sid * ROWS_PER_TILE

        if stage_feat:
            pltpu.sync_copy(htab.at[pl.ds(r0, ROWS_PER_TILE)],
                            htab_sh.at[pl.ds(r0, ROWS_PER_TILE)])

        def zacc(q, carry):
            pltpu.sync_copy(msg_v.at[0], num_sh.at[pl.ds(r0 + q * B, B)])
            if not den_in_msg:
                pltpu.sync_copy(w_v.at[0], den_sh.at[pl.ds(r0 + q * B, B)])
            return carry

        lax.fori_loop(0, ROWS_PER_TILE // B, zacc, 0)
        plsc.subcore_barrier()

        def issue_idx(i, s):
            pltpu.async_copy(src_hbm.at[pl.ds(base0 + i * B, B)],
                             sidx.at[s], isem[s])
            pltpu.async_copy(dst_hbm.at[pl.ds(base0 + i * B, B)],
                             didx.at[s], isem[s])

        def wait_idx(s):
            pltpu.make_async_copy(src_hbm.at[pl.ds(0, B)],
                                  sidx.at[s], isem[s]).wait()
            pltpu.make_async_copy(dst_hbm.at[pl.ds(0, B)],
                                  didx.at[s], isem[s]).wait()

        def issue_gath(x, s):
            pltpu.async_copy(astab.at[sidx.at[s]], as_v.at[x], gsem[x])
            pltpu.async_copy(adtab.at[didx.at[s]], ad_v.at[x], gsem[x])
            pltpu.async_copy(htab_sh.at[sidx.at[s]], h_v.at[x], gsem[x])

        def wait_gath(x, s):
            pltpu.make_async_copy(astab.at[sidx.at[s]],
                                  as_v.at[x], gsem[x]).wait()
            pltpu.make_async_copy(adtab.at[didx.at[s]],
                                  ad_v.at[x], gsem[x]).wait()
            pltpu.make_async_copy(htab_sh.at[sidx.at[s]],
                                  h_v.at[x], gsem[x]).wait()

        def issue_scat(x, s):
            if not den_in_msg:
                pltpu.async_copy(w_v.at[x], den_sh.at[didx.at[s]], ssem[x],
                                 add=True)
            pltpu.async_copy(msg_v.at[x], num_sh.at[didx.at[s]], ssem[x],
                             add=True)

        def wait_scat(x, s):
            if not den_in_msg:
                pltpu.make_async_copy(w_v.at[x], den_sh.at[didx.at[s]],
                                      ssem[x]).wait()
            pltpu.make_async_copy(msg_v.at[x], num_sh.at[didx.at[s]],
                                  ssem[x]).wait()

        def compute(x):
            @plsc.parallel_loop(0, B, unroll=4)
            def _(e):
                a = as_v[x, e, :] + ad_v[x, e, :]
                a = jnp.where(a >= 0.0, a, a * 0.2)
                w16 = jnp.exp(a)
                if not den_in_msg:
                    w_v[x, e, :] = w16
                for c in range(CH):
                    msg_v[x, e, pl.ds(c * 16, 16)] = (
                        h_v[x, e, pl.ds(c * 16, 16)] * w16)

        issue_idx(0, 0)
        issue_idx(1, 1)
        wait_idx(0)
        issue_gath(0, 0)

        def quad(k, carry):
            i0 = 4 * k
            for p in range(4):
                x = p % 2
                y = 1 - x
                sp = p
                sp1 = (p + 1) % 4
                sp2 = (p + 2) % 4
                if p < 2:
                    @pl.when(k >= 1)
                    def _():
                        wait_scat(x, sp2)
                else:
                    wait_scat(x, sp2)
                issue_idx(i0 + p + 2, sp2)
                wait_idx(sp1)
                issue_gath(y, sp1)
                wait_gath(x, sp)
                compute(x)
                issue_scat(x, sp)
            return carry

        lax.fori_loop(0, itc // 4, quad, 0)

        wait_scat(0, 2)
        wait_scat(1, 3)
        wait_gath(0, 0)
        wait_idx(1)

        plsc.subcore_barrier()
        pltpu.sync_copy(num_sh.at[pl.ds(r0, ROWS_PER_TILE)],
                        onum.at[cid, pl.ds(r0, ROWS_PER_TILE)])
        if not den_in_msg:
            pltpu.sync_copy(den_sh.at[pl.ds(r0, ROWS_PER_TILE)],
                            oden.at[cid, pl.ds(r0, ROWS_PER_TILE)])

    return sweep


_sweep1 = _sc_edge_sweep(DH1, stage_feat=False, it0=120, it1=48)
_sweep2 = _sc_edge_sweep(DH2, den_in_msg=True, stage_feat=False,
                         it0=104, it1=64)


def _tc_layer1(xp, W1p, asw, adw):
    def body(x_ref, w_ref, as_ref, ad_ref, h_ref, s_ref, d_ref):
        h = jnp.dot(x_ref[...], w_ref[...],
                    preferred_element_type=jnp.float32)
        h_ref[...] = h
        s_ref[...] = jnp.dot(h, as_ref[...],
                             preferred_element_type=jnp.float32)
        d_ref[...] = jnp.dot(h, ad_ref[...],
                             preferred_element_type=jnp.float32)

    return pl.pallas_call(
        body,
        out_shape=(
            jax.ShapeDtypeStruct((NPAD, DH1), jnp.float32),
            jax.ShapeDtypeStruct((NPAD, 16), jnp.float32),
            jax.ShapeDtypeStruct((NPAD, 16), jnp.float32),
        ),
    )(xp, W1p, asw, adw)


def _tc_combine1(onum, oden, b1r, W2p, as2w, ad2w, rmat, one10):
    def body(on_ref, od_ref, b1_ref, w2_ref, as_ref, ad_ref, r_ref,
             one_ref, h2_ref, s2_ref, d2_ref):
        o = on_ref[0] + on_ref[1]
        den = od_ref[0] + od_ref[1]
        inv = 1.0 / (den + 1e-16)
        invb = jnp.dot(inv, r_ref[...], preferred_element_type=jnp.float32)
        out1 = o * invb + b1_ref[...]
        x2 = jnp.where(out1 > 0.0, out1, jnp.exp(out1) - 1.0)
        h2 = jnp.dot(x2, w2_ref[...], preferred_element_type=jnp.float32)
        h2_ref[...] = h2 + one_ref[...]
        s2_ref[...] = jnp.dot(h2, as_ref[...],
                              preferred_element_type=jnp.float32)
        d2_ref[...] = jnp.dot(h2, ad_ref[...],
                              preferred_element_type=jnp.float32)

    return pl.pallas_call(
        body,
        out_shape=(
            jax.ShapeDtypeStruct((NPAD, DH2), jnp.float32),
            jax.ShapeDtypeStruct((NPAD, 16), jnp.float32),
            jax.ShapeDtypeStruct((NPAD, 16), jnp.float32),
        ),
    )(onum, oden, b1r, W2p, as2w, ad2w, rmat, one10)


def _tc_final(onum, b2r):
    def body(on_ref, b2_ref, out_ref):
        o = on_ref[0] + on_ref[1]
        col = lax.broadcasted_iota(jnp.int32, (NPAD, 16), 1)
        d0 = jnp.sum(jnp.where(col == 10, o, 0.0), axis=1, keepdims=True)
        logits = o * (1.0 / (d0 + 1e-16)) + b2_ref[...]
        valid = col < C
        masked = jnp.where(valid, logits, -jnp.inf)
        m = jnp.max(masked, axis=1, keepdims=True)
        z = logits - m
        ez = jnp.where(valid, jnp.exp(z), 0.0)
        s = jnp.sum(ez, axis=1, keepdims=True)
        out_ref[...] = z - jnp.log(s)

    return pl.pallas_call(
        body,
        out_shape=jax.ShapeDtypeStruct((NPAD, 16), jnp.float32),
    )(onum, b2r)


def kernel(x, edge_index, W1, att_src1, att_dst1, b1,
           W2, att_src2, att_dst2, b2):
    loop = jnp.arange(N, dtype=jnp.int32)
    npadedge = EARR - ET
    src = jnp.concatenate([edge_index[0].astype(jnp.int32), loop,
                           jnp.zeros((npadedge,), jnp.int32)])
    dst = jnp.concatenate([edge_index[1].astype(jnp.int32), loop,
                           jnp.full((npadedge,), N, jnp.int32)])

    xp = jnp.zeros((NPAD, F_IN), jnp.float32).at[:N].set(x)

    j = jnp.arange(DH1)
    perm = (j % H1) * D1 + j // H1
    W1p = W1[:, perm]
    hh = j % H1
    dd = j // H1
    kk = jnp.arange(16)
    sel = (hh[:, None] == (kk[None, :] % H1)).astype(jnp.float32)
    asw = att_src1[hh, dd][:, None] * sel
    adw = att_dst1[hh, dd][:, None] * sel
    rmat = jnp.zeros((16, DH1), jnp.float32).at[j % H1, j].set(1.0)
    b1r = b1[perm].reshape(1, DH1)
    W2p = jnp.zeros((DH1, 16), jnp.float32).at[:, :C].set(W2[perm, :])
    as2w = jnp.zeros((16, 16), jnp.float32).at[:C, :].set(
        jnp.broadcast_to(att_src2[0][:, None], (C, 16)))
    ad2w = jnp.zeros((16, 16), jnp.float32).at[:C, :].set(
        jnp.broadcast_to(att_dst2[0][:, None], (C, 16)))
    b2r = jnp.zeros((1, 16), jnp.float32).at[0, :C].set(b2)
    one10 = jnp.zeros((1, 16), jnp.float32).at[0, C].set(1.0)

    h1, as1, ad1 = _tc_layer1(xp, W1p, asw, adw)
    onum1, oden1 = _sweep1(src, dst, h1, as1, ad1)

    h2, as2, ad2 = _tc_combine1(onum1, oden1, b1r, W2p, as2w, ad2w, rmat,
                                one10)
    onum2 = _sweep2(src, dst, h2, as2, ad2)

    out = _tc_final(onum2, b2r)
    return out[:N, :C]
